# baseline (speedup 1.0000x reference)
import jax
import jax.numpy as jnp
from jax import lax
from jax.experimental import pallas as pl
from jax.experimental.pallas import tpu as pltpu
from jax.experimental.pallas import tpu_sc as plsc

_C, _AUXC, _H, _W = 256, 64, 64, 64
_EMB = 64
_WS = 7
_PS = 5
_K = 12
_N = _H * _W
_NWIN = _WS * _WS
_CHA = 1024
_CHB = 512
_NA = _N // _CHA
_NB = _N // _CHB


def _offsets(radius):
    return [(di, dj, di * _W + dj)
            for di in range(-radius, radius + 1)
            for dj in range(-radius, radius + 1)]


def _halo_specs(nchan, ch, nblk):
    return [
        pl.BlockSpec((nchan, ch), lambda k: (0, jnp.maximum(k - 1, 0))),
        pl.BlockSpec((nchan, ch), lambda k: (0, k)),
        pl.BlockSpec((nchan, ch), lambda k, n=nblk: (0, jnp.minimum(k + 1, n - 1))),
    ]


def _mask(pg, dj, o):
    wcol = pg & (_W - 1)
    return ((wcol + dj >= 0) & (wcol + dj < _W)
            & (pg + o >= 0) & (pg + o < _N))


def _conv_body(auxl_ref, auxc_ref, auxr_ref, Wr_ref, Wphi_ref,
               Wth_ref, phi_ref, th_ref):
    f32 = jnp.float32
    pg = (jax.lax.broadcasted_iota(jnp.int32, (1, _CHA), 1)
          + pl.program_id(0) * _CHA)
    loc = jnp.concatenate([auxl_ref[:], auxc_ref[:], auxr_ref[:]], axis=1)
    acc = jnp.zeros((_EMB, _CHA), f32)
    for i in range(_PS):
        for j in range(_PS):
            di, dj = i - _PS // 2, j - _PS // 2
            o = di * _W + dj
            m = _mask(pg, dj, o).astype(f32)
            sl = loc[:, _CHA + o:2 * _CHA + o]
            acc = acc + jnp.dot(Wr_ref[i * _PS + j], sl * m,
                                preferred_element_type=f32)
    phi_ref[:] = jnp.dot(Wphi_ref[:], acc, preferred_element_type=f32)
    th_ref[:] = jnp.dot(Wth_ref[:], acc, preferred_element_type=f32)


def _att_body(phi_ref, thl_ref, thc_ref, thr_ref, att_ref):
    f32 = jnp.float32
    pg = (jax.lax.broadcasted_iota(jnp.int32, (1, _CHA), 1)
          + pl.program_id(0) * _CHA)
    loc = jnp.concatenate([thl_ref[:], thc_ref[:], thr_ref[:]], axis=1)
    phi = phi_ref[:]
    atts = []
    for di, dj, o in _offsets(_WS // 2):
        sl = loc[:, _CHA + o:2 * _CHA + o]
        a = jnp.sum(phi * sl, axis=0, keepdims=True)
        atts.append(jnp.where(_mask(pg, dj, o), a, 0.0))
    atts.append(jnp.zeros((64 - _NWIN, _CHA), f32))
    att_ref[:] = jnp.concatenate(atts, axis=0)


_WPW = _N // 32
_NGRP = _WPW // 16


def _sc_weights_body(att_hbm, wq_hbm, att_v, wq_v):
    f32 = jnp.float32
    wid = lax.axis_index("s") * 2 + lax.axis_index("c")
    base = wid * _WPW
    pltpu.sync_copy(att_hbm.at[:, pl.ds(base, _WPW)], att_v)

    lanes = lax.iota(jnp.int32, 16)
    ninf = jnp.float32(-jnp.inf)

    @plsc.parallel_loop(0, _NGRP, unroll=2)
    def group(gi):
        g16 = gi * 16
        pidx = base + g16 + lanes
        wcol = pidx & (_W - 1)
        atts = [att_v[o, pl.ds(g16, 16)] for o in range(_NWIN)]

        m0 = atts[0]
        for o in range(1, _NWIN):
            m0 = jnp.maximum(m0, atts[o])

        def kth(_, tc):
            t, cnt = tc
            nv = jnp.full((16,), ninf, f32)
            for o in range(_NWIN):
                nv = jnp.maximum(nv, jnp.where(atts[o] < t, atts[o], ninf))
            c = jnp.zeros((16,), f32)
            for o in range(_NWIN):
                c = c + jnp.where(atts[o] == nv, 1.0, 0.0)
            active = cnt < _K
            return (jnp.where(active, nv, t), jnp.where(active, cnt + c, cnt))
        t, cnt = lax.fori_loop(0, _K, kth, (jnp.full((16,), jnp.inf, f32),
                                            jnp.zeros((16,), f32)))

        ngt = jnp.zeros((16,), f32)
        for o in range(_NWIN):
            ngt = ngt + jnp.where(atts[o] > t, 1.0, 0.0)
        neq = _K - ngt
        et = jnp.exp(t - m0)

        pc = jnp.zeros((16,), f32)
        den = jnp.zeros((16,), f32)
        oi = 0
        for di in range(-3, 4):
            for dj in range(-3, 4):
                off = di * _W + dj
                a = atts[oi]
                eq = a == t
                wv = jnp.where(a > t, jnp.exp(a - m0), 0.0) \
                    + jnp.where(eq & (pc < neq), et, 0.0)
                den = den + wv
                pc = pc + jnp.where(eq, 1.0, 0.0)
                valid = ((wcol + dj >= 0) & (wcol + dj < _W)
                         & (pidx + off >= 0) & (pidx + off < _N))
                wq_v[oi, pl.ds(g16, 16)] = jnp.where(valid, wv, 0.0)
                oi += 1
        rden = 1.0 / den
        for o in range(_NWIN):
            wq_v[o, pl.ds(g16, 16)] = wq_v[o, pl.ds(g16, 16)] * rden

    pltpu.sync_copy(wq_v, wq_hbm.at[:, pl.ds(base, _WPW)])


def _g_body(u_ref, Wg_ref, g_ref):
    g_ref[:] = jnp.dot(Wg_ref[:], u_ref[:], preferred_element_type=jnp.float32)


_PB = 4


def _comb_body(wl_ref, wc_ref, wr_ref, gl_ref, gc_ref, gr_ref, out_ref):
    f32 = jnp.float32
    loc = jnp.concatenate([gl_ref[:], gc_ref[:], gr_ref[:]], axis=1)
    wloc = jnp.concatenate([wl_ref[:], wc_ref[:], wr_ref[:]], axis=1)
    gs = {di: loc[:, _CHB - _PB + di * _W:2 * _CHB + _PB + di * _W]
          for di in range(-3, 4)}
    acc = jnp.zeros((_C, _CHB), f32)
    for dj in range(-3, 4):
        bdj = jnp.zeros((_C, _CHB + 2 * _PB), f32)
        for di in range(-3, 4):
            oi = (di + 3) * _WS + (dj + 3)
            wrow = wloc[oi:oi + 1, _CHB - _PB - dj:2 * _CHB + _PB - dj]
            bdj = bdj + wrow * gs[di]
        acc = acc + bdj[:, _PB + dj:_PB + dj + _CHB]
    out_ref[:] = acc


def kernel(u, aux, W_emb, W_g, W_phi, W_theta):
    b = u.shape[0]
    f32 = jnp.float32
    u2 = u.reshape(_C, _N)
    aux2 = aux.reshape(_AUXC, _N)
    Wr = W_emb.reshape(_EMB, _AUXC, _PS * _PS).transpose(2, 0, 1)

    full = lambda s: pl.BlockSpec(s, lambda k: (0,) * len(s))
    par = pltpu.CompilerParams(dimension_semantics=("parallel",))

    phi, th = pl.pallas_call(
        _conv_body,
        grid=(_NA,),
        in_specs=[*_halo_specs(_AUXC, _CHA, _NA),
                  full((_PS * _PS, _EMB, _AUXC)),
                  full((_EMB, _EMB)), full((_EMB, _EMB))],
        out_specs=[pl.BlockSpec((_EMB, _CHA), lambda k: (0, k))] * 2,
        out_shape=[jax.ShapeDtypeStruct((_EMB, _N), f32)] * 2,
        compiler_params=par,
    )(aux2, aux2, aux2, Wr, W_phi, W_theta)

    att64 = pl.pallas_call(
        _att_body,
        grid=(_NA,),
        in_specs=[pl.BlockSpec((_EMB, _CHA), lambda k: (0, k)),
                  *_halo_specs(_EMB, _CHA, _NA)],
        out_specs=pl.BlockSpec((64, _CHA), lambda k: (0, k)),
        out_shape=jax.ShapeDtypeStruct((64, _N), f32),
        compiler_params=par,
    )(phi, th, th, th)

    wq = pl.kernel(
        _sc_weights_body,
        out_type=jax.ShapeDtypeStruct((64, _N), f32),
        mesh=plsc.VectorSubcoreMesh(core_axis_name="c", subcore_axis_name="s"),
        scratch_types=[pltpu.VMEM((64, _WPW), f32),
                       pltpu.VMEM((64, _WPW), f32)],
    )(att64)

    g = pl.pallas_call(
        _g_body,
        grid=(_NA,),
        in_specs=[pl.BlockSpec((_C, _CHA), lambda k: (0, k)),
                  full((_C, _C))],
        out_specs=pl.BlockSpec((_C, _CHA), lambda k: (0, k)),
        out_shape=jax.ShapeDtypeStruct((_C, _N), f32),
        compiler_params=par,
    )(u2, W_g)

    out = pl.pallas_call(
        _comb_body,
        grid=(_NB,),
        in_specs=[*_halo_specs(64, _CHB, _NB),
                  *_halo_specs(_C, _CHB, _NB)],
        out_specs=pl.BlockSpec((_C, _CHB), lambda k: (0, k)),
        out_shape=jax.ShapeDtypeStruct((_C, _N), f32),
        compiler_params=par,
    )(wq, wq, wq, g, g, g)

    return out.reshape(b, _C, _H, _W)


# speedup vs baseline: 1.0895x; 1.0895x over previous
import jax
import jax.numpy as jnp
from jax import lax
from jax.experimental import pallas as pl
from jax.experimental.pallas import tpu as pltpu
from jax.experimental.pallas import tpu_sc as plsc

_C, _AUXC, _H, _W = 256, 64, 64, 64
_EMB = 64
_WS = 7
_PS = 5
_K = 12
_N = _H * _W
_NWIN = _WS * _WS
_CHA = 1024
_CHB = 512
_NA = _N // _CHA
_NB = _N // _CHB


def _offsets(radius):
    return [(di, dj, di * _W + dj)
            for di in range(-radius, radius + 1)
            for dj in range(-radius, radius + 1)]


def _halo_specs(nchan, ch, nblk):
    return [
        pl.BlockSpec((nchan, ch), lambda k: (0, jnp.maximum(k - 1, 0))),
        pl.BlockSpec((nchan, ch), lambda k: (0, k)),
        pl.BlockSpec((nchan, ch), lambda k, n=nblk: (0, jnp.minimum(k + 1, n - 1))),
    ]


def _mask(pg, dj, o):
    wcol = pg & (_W - 1)
    return ((wcol + dj >= 0) & (wcol + dj < _W)
            & (pg + o >= 0) & (pg + o < _N))


def _conv_body(auxl_ref, auxc_ref, auxr_ref, Wr_ref, Wphi_ref,
               Wth_ref, phi_ref, th_ref):
    f32 = jnp.float32
    pg = (jax.lax.broadcasted_iota(jnp.int32, (1, _CHA), 1)
          + pl.program_id(0) * _CHA)
    loc = jnp.concatenate([auxl_ref[:], auxc_ref[:], auxr_ref[:]], axis=1)
    acc = jnp.zeros((_EMB, _CHA), f32)
    for i in range(_PS):
        for j in range(_PS):
            di, dj = i - _PS // 2, j - _PS // 2
            o = di * _W + dj
            m = _mask(pg, dj, o).astype(f32)
            sl = loc[:, _CHA + o:2 * _CHA + o]
            acc = acc + jnp.dot(Wr_ref[i * _PS + j], sl * m,
                                preferred_element_type=f32)
    phi_ref[:] = jnp.dot(Wphi_ref[:], acc, preferred_element_type=f32)
    th_ref[:] = jnp.dot(Wth_ref[:], acc, preferred_element_type=f32)


def _att_body(phi_ref, thl_ref, thc_ref, thr_ref, att_ref):
    f32 = jnp.float32
    pg = (jax.lax.broadcasted_iota(jnp.int32, (1, _CHA), 1)
          + pl.program_id(0) * _CHA)
    loc = jnp.concatenate([thl_ref[:], thc_ref[:], thr_ref[:]], axis=1)
    phi = phi_ref[:]
    atts = []
    for di, dj, o in _offsets(_WS // 2):
        sl = loc[:, _CHA + o:2 * _CHA + o]
        a = jnp.sum(phi * sl, axis=0, keepdims=True)
        atts.append(jnp.where(_mask(pg, dj, o), a, 0.0))
    atts.append(jnp.zeros((64 - _NWIN, _CHA), f32))
    att_ref[:] = jnp.concatenate(atts, axis=0)


_WPW = _N // 32
_NGRP = _WPW // 16


def _sc_weights_body(att_hbm, wq_hbm, att_v, wq_v):
    f32 = jnp.float32
    wid = lax.axis_index("s") * 2 + lax.axis_index("c")
    base = wid * _WPW
    pltpu.sync_copy(att_hbm.at[:, pl.ds(base, _WPW)], att_v)

    lanes = lax.iota(jnp.int32, 16)
    ninf = jnp.float32(-jnp.inf)

    @plsc.parallel_loop(0, _NGRP)
    def group(gi):
        g16 = gi * 16
        pidx = base + g16 + lanes
        wcol = pidx & (_W - 1)
        atts = [att_v[o, pl.ds(g16, 16)] for o in range(_NWIN)]

        m0 = atts[0]
        for o in range(1, _NWIN):
            m0 = jnp.maximum(m0, atts[o])

        def kth(_, tc):
            t, cnt = tc
            nv = jnp.full((16,), ninf, f32)
            for o in range(_NWIN):
                nv = jnp.maximum(nv, jnp.where(atts[o] < t, atts[o], ninf))
            c = jnp.zeros((16,), f32)
            for o in range(_NWIN):
                c = c + jnp.where(atts[o] == nv, 1.0, 0.0)
            active = cnt < _K
            return (jnp.where(active, nv, t), jnp.where(active, cnt + c, cnt))
        t, cnt = lax.fori_loop(0, _K, kth, (jnp.full((16,), jnp.inf, f32),
                                            jnp.zeros((16,), f32)))

        ngt = jnp.zeros((16,), f32)
        for o in range(_NWIN):
            ngt = ngt + jnp.where(atts[o] > t, 1.0, 0.0)
        neq = _K - ngt
        et = jnp.exp(t - m0)

        pc = jnp.zeros((16,), f32)
        den = jnp.zeros((16,), f32)
        oi = 0
        for di in range(-3, 4):
            for dj in range(-3, 4):
                off = di * _W + dj
                a = atts[oi]
                eq = a == t
                wv = jnp.where(a > t, jnp.exp(a - m0), 0.0) \
                    + jnp.where(eq & (pc < neq), et, 0.0)
                den = den + wv
                pc = pc + jnp.where(eq, 1.0, 0.0)
                valid = ((wcol + dj >= 0) & (wcol + dj < _W)
                         & (pidx + off >= 0) & (pidx + off < _N))
                wq_v[oi, pl.ds(g16, 16)] = jnp.where(valid, wv, 0.0)
                oi += 1
        rden = 1.0 / den
        for o in range(_NWIN):
            wq_v[o, pl.ds(g16, 16)] = wq_v[o, pl.ds(g16, 16)] * rden

    pltpu.sync_copy(wq_v, wq_hbm.at[:, pl.ds(base, _WPW)])


def _g_body(u_ref, Wg_ref, g_ref):
    g_ref[:] = jnp.dot(Wg_ref[:], u_ref[:], preferred_element_type=jnp.float32)


_PB = 4


def _comb_body(wl_ref, wc_ref, wr_ref, gl_ref, gc_ref, gr_ref, out_ref):
    f32 = jnp.float32
    loc = jnp.concatenate([gl_ref[:], gc_ref[:], gr_ref[:]], axis=1)
    wloc = jnp.concatenate([wl_ref[:], wc_ref[:], wr_ref[:]], axis=1)
    gs = {di: loc[:, _CHB - _PB + di * _W:2 * _CHB + _PB + di * _W]
          for di in range(-3, 4)}
    acc = jnp.zeros((_C, _CHB), f32)
    for dj in range(-3, 4):
        bdj = jnp.zeros((_C, _CHB + 2 * _PB), f32)
        for di in range(-3, 4):
            oi = (di + 3) * _WS + (dj + 3)
            wrow = wloc[oi:oi + 1, _CHB - _PB - dj:2 * _CHB + _PB - dj]
            bdj = bdj + wrow * gs[di]
        acc = acc + bdj[:, _PB + dj:_PB + dj + _CHB]
    out_ref[:] = acc


def kernel(u, aux, W_emb, W_g, W_phi, W_theta):
    b = u.shape[0]
    f32 = jnp.float32
    u2 = u.reshape(_C, _N)
    aux2 = aux.reshape(_AUXC, _N)
    Wr = W_emb.reshape(_EMB, _AUXC, _PS * _PS).transpose(2, 0, 1)

    full = lambda s: pl.BlockSpec(s, lambda k: (0,) * len(s))
    par = pltpu.CompilerParams(dimension_semantics=("parallel",))

    phi, th = pl.pallas_call(
        _conv_body,
        grid=(_NA,),
        in_specs=[*_halo_specs(_AUXC, _CHA, _NA),
                  full((_PS * _PS, _EMB, _AUXC)),
                  full((_EMB, _EMB)), full((_EMB, _EMB))],
        out_specs=[pl.BlockSpec((_EMB, _CHA), lambda k: (0, k))] * 2,
        out_shape=[jax.ShapeDtypeStruct((_EMB, _N), f32)] * 2,
        compiler_params=par,
    )(aux2, aux2, aux2, Wr, W_phi, W_theta)

    att64 = pl.pallas_call(
        _att_body,
        grid=(_NA,),
        in_specs=[pl.BlockSpec((_EMB, _CHA), lambda k: (0, k)),
                  *_halo_specs(_EMB, _CHA, _NA)],
        out_specs=pl.BlockSpec((64, _CHA), lambda k: (0, k)),
        out_shape=jax.ShapeDtypeStruct((64, _N), f32),
        compiler_params=par,
    )(phi, th, th, th)

    wq = pl.kernel(
        _sc_weights_body,
        out_type=jax.ShapeDtypeStruct((64, _N), f32),
        mesh=plsc.VectorSubcoreMesh(core_axis_name="c", subcore_axis_name="s"),
        scratch_types=[pltpu.VMEM((64, _WPW), f32),
                       pltpu.VMEM((64, _WPW), f32)],
    )(att64)

    g = pl.pallas_call(
        _g_body,
        grid=(_NA,),
        in_specs=[pl.BlockSpec((_C, _CHA), lambda k: (0, k)),
                  full((_C, _C))],
        out_specs=pl.BlockSpec((_C, _CHA), lambda k: (0, k)),
        out_shape=jax.ShapeDtypeStruct((_C, _N), f32),
        compiler_params=par,
    )(u2, W_g)

    out = pl.pallas_call(
        _comb_body,
        grid=(_NB,),
        in_specs=[*_halo_specs(64, _CHB, _NB),
                  *_halo_specs(_C, _CHB, _NB)],
        out_specs=pl.BlockSpec((_C, _CHB), lambda k: (0, k)),
        out_shape=jax.ShapeDtypeStruct((_C, _N), f32),
        compiler_params=par,
    )(wq, wq, wq, g, g, g)

    return out.reshape(b, _C, _H, _W)


# speedup vs baseline: 1.1323x; 1.0393x over previous
import jax
import jax.numpy as jnp
from jax import lax
from jax.experimental import pallas as pl
from jax.experimental.pallas import tpu as pltpu
from jax.experimental.pallas import tpu_sc as plsc

_C, _AUXC, _H, _W = 256, 64, 64, 64
_EMB = 64
_WS = 7
_PS = 5
_K = 12
_N = _H * _W
_NWIN = _WS * _WS
_CHA = 1024
_CHB = 512
_NA = _N // _CHA
_NB = _N // _CHB


def _offsets(radius):
    return [(di, dj, di * _W + dj)
            for di in range(-radius, radius + 1)
            for dj in range(-radius, radius + 1)]


def _halo_specs(nchan, ch, nblk):
    return [
        pl.BlockSpec((nchan, ch), lambda k: (0, jnp.maximum(k - 1, 0))),
        pl.BlockSpec((nchan, ch), lambda k: (0, k)),
        pl.BlockSpec((nchan, ch), lambda k, n=nblk: (0, jnp.minimum(k + 1, n - 1))),
    ]


def _mask(pg, dj, o):
    wcol = pg & (_W - 1)
    return ((wcol + dj >= 0) & (wcol + dj < _W)
            & (pg + o >= 0) & (pg + o < _N))


def _conv_body(auxl_ref, auxc_ref, auxr_ref, Wr_ref, Wphi_ref,
               Wth_ref, phi_ref, th_ref):
    f32 = jnp.float32
    pg = (jax.lax.broadcasted_iota(jnp.int32, (1, _CHA), 1)
          + pl.program_id(0) * _CHA)
    loc = jnp.concatenate([auxl_ref[:], auxc_ref[:], auxr_ref[:]], axis=1)
    acc = jnp.zeros((_EMB, _CHA), f32)
    for i in range(_PS):
        for j in range(_PS):
            di, dj = i - _PS // 2, j - _PS // 2
            o = di * _W + dj
            m = _mask(pg, dj, o).astype(f32)
            sl = loc[:, _CHA + o:2 * _CHA + o]
            acc = acc + jnp.dot(Wr_ref[i * _PS + j], sl * m,
                                preferred_element_type=f32)
    phi_ref[:] = jnp.dot(Wphi_ref[:], acc, preferred_element_type=f32)
    th_ref[:] = jnp.dot(Wth_ref[:], acc, preferred_element_type=f32)


def _att_body(phi_ref, thl_ref, thc_ref, thr_ref, att_ref):
    f32 = jnp.float32
    pg = (jax.lax.broadcasted_iota(jnp.int32, (1, _CHA), 1)
          + pl.program_id(0) * _CHA)
    loc = jnp.concatenate([thl_ref[:], thc_ref[:], thr_ref[:]], axis=1)
    phi = phi_ref[:]
    atts = []
    for di, dj, o in _offsets(_WS // 2):
        sl = loc[:, _CHA + o:2 * _CHA + o]
        a = jnp.sum(phi * sl, axis=0, keepdims=True)
        atts.append(jnp.where(_mask(pg, dj, o), a, 0.0))
    atts.append(jnp.zeros((64 - _NWIN, _CHA), f32))
    att_ref[:] = jnp.concatenate(atts, axis=0)


_WPW = _N // 32
_NGRP = _WPW // 16


def _sc_weights_body(att_hbm, wq_hbm, att_v, wq_v):
    f32 = jnp.float32
    wid = lax.axis_index("s") * 2 + lax.axis_index("c")
    base = wid * _WPW
    pltpu.sync_copy(att_hbm.at[:, pl.ds(base, _WPW)], att_v)

    lanes = lax.iota(jnp.int32, 16)
    ninf = jnp.float32(-jnp.inf)

    @plsc.parallel_loop(0, _NGRP)
    def group(gi):
        g16 = gi * 16
        pidx = base + g16 + lanes
        wcol = pidx & (_W - 1)
        atts = [att_v[o, pl.ds(g16, 16)] for o in range(_NWIN)]

        m0 = atts[0]
        for o in range(1, _NWIN):
            m0 = jnp.maximum(m0, atts[o])

        def kth(_, tc):
            t, cnt = tc
            nv = jnp.full((16,), ninf, f32)
            for o in range(_NWIN):
                nv = jnp.maximum(nv, jnp.where(atts[o] < t, atts[o], ninf))
            c = jnp.zeros((16,), f32)
            for o in range(_NWIN):
                c = c + jnp.where(atts[o] == nv, 1.0, 0.0)
            active = cnt < _K
            return (jnp.where(active, nv, t), jnp.where(active, cnt + c, cnt))
        t, cnt = lax.fori_loop(0, _K, kth, (jnp.full((16,), jnp.inf, f32),
                                            jnp.zeros((16,), f32)))

        ngt = jnp.zeros((16,), f32)
        for o in range(_NWIN):
            ngt = ngt + jnp.where(atts[o] > t, 1.0, 0.0)
        neq = _K - ngt
        et = jnp.exp(t - m0)

        pc = jnp.zeros((16,), f32)
        den = jnp.zeros((16,), f32)
        oi = 0
        for di in range(-3, 4):
            for dj in range(-3, 4):
                off = di * _W + dj
                a = atts[oi]
                eq = a == t
                wv = jnp.where(a > t, jnp.exp(a - m0), 0.0) \
                    + jnp.where(eq & (pc < neq), et, 0.0)
                den = den + wv
                pc = pc + jnp.where(eq, 1.0, 0.0)
                valid = ((wcol + dj >= 0) & (wcol + dj < _W)
                         & (pidx + off >= 0) & (pidx + off < _N))
                wq_v[oi, pl.ds(g16, 16)] = jnp.where(valid, wv, 0.0)
                oi += 1
        rden = 1.0 / den
        for o in range(_NWIN):
            wq_v[o, pl.ds(g16, 16)] = wq_v[o, pl.ds(g16, 16)] * rden

    pltpu.sync_copy(wq_v, wq_hbm.at[:, pl.ds(base, _WPW)])


def _g_body(u_ref, Wg_ref, g_ref):
    g_ref[:] = jnp.dot(Wg_ref[:], u_ref[:], preferred_element_type=jnp.float32)


_PB = 4


def _comb_body(wq_ref, g_ref, out_ref):
    f32 = jnp.float32
    pad = 256
    gp = jnp.pad(g_ref[:], ((0, 0), (pad, pad)))
    wp = jnp.pad(wq_ref[:], ((0, 0), (pad, pad)))
    gs = {di: gp[:, pad - _PB + di * _W:pad + _N + _PB + di * _W]
          for di in range(-3, 4)}
    acc = jnp.zeros((_C, _N), f32)
    for dj in range(-3, 4):
        bdj = jnp.zeros((_C, _N + 2 * _PB), f32)
        for di in range(-3, 4):
            oi = (di + 3) * _WS + (dj + 3)
            wrow = wp[oi:oi + 1, pad - _PB - dj:pad + _N + _PB - dj]
            bdj = bdj + wrow * gs[di]
        acc = acc + bdj[:, _PB + dj:_PB + dj + _N]
    out_ref[:] = acc


def kernel(u, aux, W_emb, W_g, W_phi, W_theta):
    b = u.shape[0]
    f32 = jnp.float32
    u2 = u.reshape(_C, _N)
    aux2 = aux.reshape(_AUXC, _N)
    Wr = W_emb.reshape(_EMB, _AUXC, _PS * _PS).transpose(2, 0, 1)

    full = lambda s: pl.BlockSpec(s, lambda k: (0,) * len(s))
    par = pltpu.CompilerParams(dimension_semantics=("parallel",))

    phi, th = pl.pallas_call(
        _conv_body,
        grid=(_NA,),
        in_specs=[*_halo_specs(_AUXC, _CHA, _NA),
                  full((_PS * _PS, _EMB, _AUXC)),
                  full((_EMB, _EMB)), full((_EMB, _EMB))],
        out_specs=[pl.BlockSpec((_EMB, _CHA), lambda k: (0, k))] * 2,
        out_shape=[jax.ShapeDtypeStruct((_EMB, _N), f32)] * 2,
        compiler_params=par,
    )(aux2, aux2, aux2, Wr, W_phi, W_theta)

    att64 = pl.pallas_call(
        _att_body,
        grid=(_NA,),
        in_specs=[pl.BlockSpec((_EMB, _CHA), lambda k: (0, k)),
                  *_halo_specs(_EMB, _CHA, _NA)],
        out_specs=pl.BlockSpec((64, _CHA), lambda k: (0, k)),
        out_shape=jax.ShapeDtypeStruct((64, _N), f32),
        compiler_params=par,
    )(phi, th, th, th)

    wq = pl.kernel(
        _sc_weights_body,
        out_type=jax.ShapeDtypeStruct((64, _N), f32),
        mesh=plsc.VectorSubcoreMesh(core_axis_name="c", subcore_axis_name="s"),
        scratch_types=[pltpu.VMEM((64, _WPW), f32),
                       pltpu.VMEM((64, _WPW), f32)],
    )(att64)

    g = pl.pallas_call(
        _g_body,
        grid=(_NA,),
        in_specs=[pl.BlockSpec((_C, _CHA), lambda k: (0, k)),
                  full((_C, _C))],
        out_specs=pl.BlockSpec((_C, _CHA), lambda k: (0, k)),
        out_shape=jax.ShapeDtypeStruct((_C, _N), f32),
        compiler_params=par,
    )(u2, W_g)

    out = pl.pallas_call(
        _comb_body,
        out_shape=jax.ShapeDtypeStruct((_C, _N), f32),
    )(wq, g)

    return out.reshape(b, _C, _H, _W)


# speedup vs baseline: 1.2326x; 1.0886x over previous
import jax
import jax.numpy as jnp
from jax import lax
from jax.experimental import pallas as pl
from jax.experimental.pallas import tpu as pltpu
from jax.experimental.pallas import tpu_sc as plsc

_C, _AUXC, _H, _W = 256, 64, 64, 64
_EMB = 64
_WS = 7
_PS = 5
_K = 12
_N = _H * _W
_NWIN = _WS * _WS
_CHA = 1024
_CHB = 512
_NA = _N // _CHA
_NB = _N // _CHB


def _offsets(radius):
    return [(di, dj, di * _W + dj)
            for di in range(-radius, radius + 1)
            for dj in range(-radius, radius + 1)]


def _halo_specs(nchan, ch, nblk):
    return [
        pl.BlockSpec((nchan, ch), lambda k: (0, jnp.maximum(k - 1, 0))),
        pl.BlockSpec((nchan, ch), lambda k: (0, k)),
        pl.BlockSpec((nchan, ch), lambda k, n=nblk: (0, jnp.minimum(k + 1, n - 1))),
    ]


def _mask(pg, dj, o):
    wcol = pg & (_W - 1)
    return ((wcol + dj >= 0) & (wcol + dj < _W)
            & (pg + o >= 0) & (pg + o < _N))


def _convatt_body(aux_ref, Wr_ref, Wphi_ref, Wth_ref, att_ref):
    f32 = jnp.float32
    pad = 256
    pg = jax.lax.broadcasted_iota(jnp.int32, (1, _N), 1)
    auxp = jnp.pad(aux_ref[:], ((0, 0), (pad, pad)))
    acc = jnp.zeros((_EMB, _N), f32)
    for i in range(_PS):
        for j in range(_PS):
            di, dj = i - _PS // 2, j - _PS // 2
            o = di * _W + dj
            m = _mask(pg, dj, o).astype(f32)
            sl = auxp[:, pad + o:pad + _N + o]
            acc = acc + jnp.dot(Wr_ref[i * _PS + j], sl * m,
                                preferred_element_type=f32)
    phi = jnp.dot(Wphi_ref[:], acc, preferred_element_type=f32)
    th = jnp.dot(Wth_ref[:], acc, preferred_element_type=f32)
    thp = jnp.pad(th, ((0, 0), (pad, pad)))
    atts = []
    for di, dj, o in _offsets(_WS // 2):
        sl = thp[:, pad + o:pad + _N + o]
        a = jnp.sum(phi * sl, axis=0, keepdims=True)
        atts.append(jnp.where(_mask(pg, dj, o), a, 0.0))
    atts.append(jnp.zeros((64 - _NWIN, _N), f32))
    att_ref[:] = jnp.concatenate(atts, axis=0)


_WPW = _N // 32
_NGRP = _WPW // 16


def _sc_weights_body(att_hbm, wq_hbm, att_v, wq_v):
    f32 = jnp.float32
    wid = lax.axis_index("s") * 2 + lax.axis_index("c")
    base = wid * _WPW
    pltpu.sync_copy(att_hbm.at[:, pl.ds(base, _WPW)], att_v)

    lanes = lax.iota(jnp.int32, 16)
    ninf = jnp.float32(-jnp.inf)

    @plsc.parallel_loop(0, _NGRP)
    def group(gi):
        g16 = gi * 16
        pidx = base + g16 + lanes
        wcol = pidx & (_W - 1)
        atts = [att_v[o, pl.ds(g16, 16)] for o in range(_NWIN)]

        m0 = atts[0]
        for o in range(1, _NWIN):
            m0 = jnp.maximum(m0, atts[o])

        def kth(_, tc):
            t, cnt = tc
            nv = jnp.full((16,), ninf, f32)
            for o in range(_NWIN):
                nv = jnp.maximum(nv, jnp.where(atts[o] < t, atts[o], ninf))
            c = jnp.zeros((16,), f32)
            for o in range(_NWIN):
                c = c + jnp.where(atts[o] == nv, 1.0, 0.0)
            active = cnt < _K
            return (jnp.where(active, nv, t), jnp.where(active, cnt + c, cnt))
        t, cnt = lax.fori_loop(0, _K, kth, (jnp.full((16,), jnp.inf, f32),
                                            jnp.zeros((16,), f32)))

        ngt = jnp.zeros((16,), f32)
        for o in range(_NWIN):
            ngt = ngt + jnp.where(atts[o] > t, 1.0, 0.0)
        neq = _K - ngt
        et = jnp.exp(t - m0)

        pc = jnp.zeros((16,), f32)
        den = jnp.zeros((16,), f32)
        oi = 0
        for di in range(-3, 4):
            for dj in range(-3, 4):
                off = di * _W + dj
                a = atts[oi]
                eq = a == t
                wv = jnp.where(a > t, jnp.exp(a - m0), 0.0) \
                    + jnp.where(eq & (pc < neq), et, 0.0)
                den = den + wv
                pc = pc + jnp.where(eq, 1.0, 0.0)
                valid = ((wcol + dj >= 0) & (wcol + dj < _W)
                         & (pidx + off >= 0) & (pidx + off < _N))
                wq_v[oi, pl.ds(g16, 16)] = jnp.where(valid, wv, 0.0)
                oi += 1
        rden = 1.0 / den
        for o in range(_NWIN):
            wq_v[o, pl.ds(g16, 16)] = wq_v[o, pl.ds(g16, 16)] * rden

    pltpu.sync_copy(wq_v, wq_hbm.at[:, pl.ds(base, _WPW)])


def _g_body(u_ref, Wg_ref, g_ref):
    g_ref[:] = jnp.dot(Wg_ref[:], u_ref[:], preferred_element_type=jnp.float32)


_PB = 4


def _comb_body(wq_ref, g_ref, out_ref):
    f32 = jnp.float32
    pad = 256
    gp = jnp.pad(g_ref[:], ((0, 0), (pad, pad)))
    wp = jnp.pad(wq_ref[:], ((0, 0), (pad, pad)))
    gs = {di: gp[:, pad - _PB + di * _W:pad + _N + _PB + di * _W]
          for di in range(-3, 4)}
    acc = jnp.zeros((_C, _N), f32)
    for dj in range(-3, 4):
        bdj = jnp.zeros((_C, _N + 2 * _PB), f32)
        for di in range(-3, 4):
            oi = (di + 3) * _WS + (dj + 3)
            wrow = wp[oi:oi + 1, pad - _PB - dj:pad + _N + _PB - dj]
            bdj = bdj + wrow * gs[di]
        acc = acc + bdj[:, _PB + dj:_PB + dj + _N]
    out_ref[:] = acc


def kernel(u, aux, W_emb, W_g, W_phi, W_theta):
    b = u.shape[0]
    f32 = jnp.float32
    u2 = u.reshape(_C, _N)
    aux2 = aux.reshape(_AUXC, _N)
    Wr = W_emb.reshape(_EMB, _AUXC, _PS * _PS).transpose(2, 0, 1)

    full = lambda s: pl.BlockSpec(s, lambda k: (0,) * len(s))
    par = pltpu.CompilerParams(dimension_semantics=("parallel",))

    att64 = pl.pallas_call(
        _convatt_body,
        out_shape=jax.ShapeDtypeStruct((64, _N), f32),
    )(aux2, Wr, W_phi, W_theta)

    wq = pl.kernel(
        _sc_weights_body,
        out_type=jax.ShapeDtypeStruct((64, _N), f32),
        mesh=plsc.VectorSubcoreMesh(core_axis_name="c", subcore_axis_name="s"),
        scratch_types=[pltpu.VMEM((64, _WPW), f32),
                       pltpu.VMEM((64, _WPW), f32)],
    )(att64)

    g = pl.pallas_call(
        _g_body,
        grid=(_NA,),
        in_specs=[pl.BlockSpec((_C, _CHA), lambda k: (0, k)),
                  full((_C, _C))],
        out_specs=pl.BlockSpec((_C, _CHA), lambda k: (0, k)),
        out_shape=jax.ShapeDtypeStruct((_C, _N), f32),
        compiler_params=par,
    )(u2, W_g)

    out = pl.pallas_call(
        _comb_body,
        out_shape=jax.ShapeDtypeStruct((_C, _N), f32),
    )(wq, g)

    return out.reshape(b, _C, _H, _W)


# speedup vs baseline: 1.2621x; 1.0239x over previous
import jax
import jax.numpy as jnp
from jax import lax
from jax.experimental import pallas as pl
from jax.experimental.pallas import tpu as pltpu
from jax.experimental.pallas import tpu_sc as plsc

_C, _AUXC, _H, _W = 256, 64, 64, 64
_EMB = 64
_WS = 7
_PS = 5
_K = 12
_N = _H * _W
_NWIN = _WS * _WS
_CHA = 1024
_CHB = 512
_NA = _N // _CHA
_NB = _N // _CHB


def _offsets(radius):
    return [(di, dj, di * _W + dj)
            for di in range(-radius, radius + 1)
            for dj in range(-radius, radius + 1)]


def _halo_specs(nchan, ch, nblk):
    return [
        pl.BlockSpec((nchan, ch), lambda k: (0, jnp.maximum(k - 1, 0))),
        pl.BlockSpec((nchan, ch), lambda k: (0, k)),
        pl.BlockSpec((nchan, ch), lambda k, n=nblk: (0, jnp.minimum(k + 1, n - 1))),
    ]


def _mask(pg, dj, o):
    wcol = pg & (_W - 1)
    return ((wcol + dj >= 0) & (wcol + dj < _W)
            & (pg + o >= 0) & (pg + o < _N))


def _convatt_body(aux_ref, Wr_ref, Wphi_ref, Wth_ref, att_ref):
    f32 = jnp.float32
    pad = 256
    pg = jax.lax.broadcasted_iota(jnp.int32, (1, _N), 1)
    auxp = jnp.pad(aux_ref[:], ((0, 0), (pad, pad)))
    wcq = (jax.lax.broadcasted_iota(jnp.int32, (1, _N + 2 * pad), 1)
           - pad) & (_W - 1)
    auxm = {0: auxp}
    for dj in range(-(_PS // 2), _PS // 2 + 1):
        if dj:
            ok = (wcq >= max(0, dj)) & (wcq < _W + min(0, dj))
            auxm[dj] = auxp * ok.astype(f32)
    acc = jnp.zeros((_EMB, _N), f32)
    for i in range(_PS):
        for j in range(_PS):
            di, dj = i - _PS // 2, j - _PS // 2
            o = di * _W + dj
            sl = auxm[dj][:, pad + o:pad + _N + o]
            acc = acc + jnp.dot(Wr_ref[i * _PS + j], sl,
                                preferred_element_type=f32)
    phi = jnp.dot(Wphi_ref[:], acc, preferred_element_type=f32)
    th = jnp.dot(Wth_ref[:], acc, preferred_element_type=f32)
    thp = jnp.pad(th, ((0, 0), (pad, pad)))
    atts = []
    for di, dj, o in _offsets(_WS // 2):
        sl = thp[:, pad + o:pad + _N + o]
        a = jnp.sum(phi * sl, axis=0, keepdims=True)
        atts.append(jnp.where(_mask(pg, dj, o), a, 0.0))
    atts.append(jnp.zeros((64 - _NWIN, _N), f32))
    att_ref[:] = jnp.concatenate(atts, axis=0)


_WPW = _N // 32
_NGRP = _WPW // 16


def _sc_weights_body(att_hbm, wq_hbm, att_v, wq_v):
    f32 = jnp.float32
    wid = lax.axis_index("s") * 2 + lax.axis_index("c")
    base = wid * _WPW
    pltpu.sync_copy(att_hbm.at[:, pl.ds(base, _WPW)], att_v)

    lanes = lax.iota(jnp.int32, 16)
    ninf = jnp.float32(-jnp.inf)

    @plsc.parallel_loop(0, _NGRP)
    def group(gi):
        g16 = gi * 16
        pidx = base + g16 + lanes
        wcol = pidx & (_W - 1)
        atts = [att_v[o, pl.ds(g16, 16)] for o in range(_NWIN)]

        m0 = atts[0]
        for o in range(1, _NWIN):
            m0 = jnp.maximum(m0, atts[o])

        def kth(_, tc):
            t, cnt = tc
            nv = jnp.full((16,), ninf, f32)
            for o in range(_NWIN):
                nv = jnp.maximum(nv, jnp.where(atts[o] < t, atts[o], ninf))
            c = jnp.zeros((16,), f32)
            for o in range(_NWIN):
                c = c + jnp.where(atts[o] == nv, 1.0, 0.0)
            active = cnt < _K
            return (jnp.where(active, nv, t), jnp.where(active, cnt + c, cnt))
        t, cnt = lax.fori_loop(0, _K, kth, (jnp.full((16,), jnp.inf, f32),
                                            jnp.zeros((16,), f32)))

        ngt = jnp.zeros((16,), f32)
        for o in range(_NWIN):
            ngt = ngt + jnp.where(atts[o] > t, 1.0, 0.0)
        neq = _K - ngt
        et = jnp.exp(t - m0)

        pc = jnp.zeros((16,), f32)
        den = jnp.zeros((16,), f32)
        oi = 0
        for di in range(-3, 4):
            for dj in range(-3, 4):
                off = di * _W + dj
                a = atts[oi]
                eq = a == t
                wv = jnp.where(a > t, jnp.exp(a - m0), 0.0) \
                    + jnp.where(eq & (pc < neq), et, 0.0)
                den = den + wv
                pc = pc + jnp.where(eq, 1.0, 0.0)
                valid = ((wcol + dj >= 0) & (wcol + dj < _W)
                         & (pidx + off >= 0) & (pidx + off < _N))
                wq_v[oi, pl.ds(g16, 16)] = jnp.where(valid, wv, 0.0)
                oi += 1
        rden = 1.0 / den
        for o in range(_NWIN):
            wq_v[o, pl.ds(g16, 16)] = wq_v[o, pl.ds(g16, 16)] * rden

    pltpu.sync_copy(wq_v, wq_hbm.at[:, pl.ds(base, _WPW)])


def _g_body(u_ref, Wg_ref, g_ref):
    g_ref[:] = jnp.dot(Wg_ref[:], u_ref[:], preferred_element_type=jnp.float32)


_PB = 4


def _comb_body(wq_ref, g_ref, out_ref):
    f32 = jnp.float32
    pad = 256
    gp = jnp.pad(g_ref[:], ((0, 0), (pad, pad)))
    wp = jnp.pad(wq_ref[:], ((0, 0), (pad, pad)))
    gs = {di: gp[:, pad - _PB + di * _W:pad + _N + _PB + di * _W]
          for di in range(-3, 4)}
    acc = jnp.zeros((_C, _N), f32)
    for dj in range(-3, 4):
        bdj = jnp.zeros((_C, _N + 2 * _PB), f32)
        for di in range(-3, 4):
            oi = (di + 3) * _WS + (dj + 3)
            wrow = wp[oi:oi + 1, pad - _PB - dj:pad + _N + _PB - dj]
            bdj = bdj + wrow * gs[di]
        acc = acc + bdj[:, _PB + dj:_PB + dj + _N]
    out_ref[:] = acc


def kernel(u, aux, W_emb, W_g, W_phi, W_theta):
    b = u.shape[0]
    f32 = jnp.float32
    u2 = u.reshape(_C, _N)
    aux2 = aux.reshape(_AUXC, _N)
    Wr = W_emb.reshape(_EMB, _AUXC, _PS * _PS).transpose(2, 0, 1)

    full = lambda s: pl.BlockSpec(s, lambda k: (0,) * len(s))
    par = pltpu.CompilerParams(dimension_semantics=("parallel",))

    att64 = pl.pallas_call(
        _convatt_body,
        out_shape=jax.ShapeDtypeStruct((64, _N), f32),
    )(aux2, Wr, W_phi, W_theta)

    wq = pl.kernel(
        _sc_weights_body,
        out_type=jax.ShapeDtypeStruct((64, _N), f32),
        mesh=plsc.VectorSubcoreMesh(core_axis_name="c", subcore_axis_name="s"),
        scratch_types=[pltpu.VMEM((64, _WPW), f32),
                       pltpu.VMEM((64, _WPW), f32)],
    )(att64)

    g = pl.pallas_call(
        _g_body,
        grid=(_NA,),
        in_specs=[pl.BlockSpec((_C, _CHA), lambda k: (0, k)),
                  full((_C, _C))],
        out_specs=pl.BlockSpec((_C, _CHA), lambda k: (0, k)),
        out_shape=jax.ShapeDtypeStruct((_C, _N), f32),
        compiler_params=par,
    )(u2, W_g)

    out = pl.pallas_call(
        _comb_body,
        out_shape=jax.ShapeDtypeStruct((_C, _N), f32),
    )(wq, g)

    return out.reshape(b, _C, _H, _W)
